# Initial kernel scaffold; baseline (speedup 1.0000x reference)
import jax
import jax.numpy as jnp
from jax.experimental import pallas as pl


def kernel(x, l, W3, W4, W5, W6, W7, W8, W9, W10, W11, W12, g3, b3, g4, b4, g5, b5, g6, b6, g7, b7, g8, b8, g9, b9, g10, b10, g11, b11):
    raise NotImplementedError("write your pallas kernel here")


# speedup vs baseline: 3.9893x; 3.9893x over previous
import functools

import jax
import jax.numpy as jnp
from jax import lax
from jax.experimental import pallas as pl
from jax.experimental.pallas import tpu as pltpu
from jax.experimental.pallas import tpu_sc as plsc

_K = 40


def _lrelu(v):
    return jnp.where(v >= 0, v, 0.2 * v)


def _topk_call(xt, xr, xxr, xxc, Nb=256):
    B, N, C = xt.shape
    K = _K

    def body(xt_ref, xr_ref, xxr_ref, xxc_ref, idx_ref):
        b = pl.program_id(0)
        xb = xt_ref[0]
        xN = xr_ref[0]
        inner = -2.0 * lax.dot_general(
            xb, xN, (((1,), (0,)), ((), ())),
            preferred_element_type=jnp.float32)
        vals = (-xxr_ref[0]) - inner - xxc_ref[0]
        iota = lax.broadcasted_iota(jnp.int32, (Nb, N), 1)
        kio = lax.broadcasted_iota(jnp.int32, (Nb, K), 1)
        idx_acc = jnp.zeros((Nb, K), jnp.int32)
        for j in range(K):
            m = jnp.max(vals, axis=1, keepdims=True)
            cand = jnp.where(vals == m, iota, N)
            a = jnp.min(cand, axis=1, keepdims=True)
            idx_acc = jnp.where(kio == j, a, idx_acc)
            vals = jnp.where(iota == a, -jnp.inf, vals)
        idx_ref[0] = idx_acc + b * N

    return pl.pallas_call(
        body,
        grid=(B, N // Nb),
        in_specs=[
            pl.BlockSpec((1, Nb, C), lambda b, nb: (b, nb, 0)),
            pl.BlockSpec((1, C, N), lambda b, nb: (b, 0, 0)),
            pl.BlockSpec((1, 1, N), lambda b, nb: (b, 0, 0)),
            pl.BlockSpec((1, Nb, 1), lambda b, nb: (b, nb, 0)),
        ],
        out_specs=pl.BlockSpec((1, Nb, K), lambda b, nb: (b, nb, 0)),
        out_shape=jax.ShapeDtypeStruct((B, N, K), jnp.int32),
    )(xt, xr, xxr, xxc)


def _gather_rows(table, gidx, D):
    M = gidx.shape[0]
    info = plsc.get_sparse_core_info()
    NW = info.num_cores * info.num_subcores
    bpw = M // NW
    CH = 128
    nch = bpw // CH
    mesh = plsc.VectorSubcoreMesh(core_axis_name="c", subcore_axis_name="s")

    @functools.partial(
        pl.kernel,
        mesh=mesh,
        compiler_params=pltpu.CompilerParams(use_tc_tiling_on_sc=False),
        out_type=jax.ShapeDtypeStruct((M, D), jnp.float32),
        scratch_types=[
            pltpu.VMEM((CH,), jnp.int32),
            pltpu.VMEM((CH,), jnp.int32),
            pltpu.VMEM((CH, D), jnp.float32),
            pltpu.VMEM((CH, D), jnp.float32),
            pltpu.SemaphoreType.DMA,
            pltpu.SemaphoreType.DMA,
        ],
    )
    def k(table_hbm, idx_hbm, out_hbm, idx_a, idx_b, rows_a, rows_b,
          sem_a, sem_b):
        wid = lax.axis_index("s") * info.num_cores + lax.axis_index("c")
        base = wid * bpw
        idx_bufs = (idx_a, idx_b)
        row_bufs = (rows_a, rows_b)
        sems = (sem_a, sem_b)
        pltpu.sync_copy(idx_hbm.at[pl.ds(base, CH)], idx_a)
        g_prev = pltpu.async_copy(table_hbm.at[idx_a], rows_a, sem_a)
        for c in range(nch):
            s = c % 2
            n = (c + 1) % 2
            if c + 1 < nch:
                pltpu.sync_copy(
                    idx_hbm.at[pl.ds(base + (c + 1) * CH, CH)], idx_bufs[n])
                g_next = pltpu.async_copy(
                    table_hbm.at[idx_bufs[n]], row_bufs[n], sems[n])
            g_prev.wait()
            pltpu.sync_copy(row_bufs[s], out_hbm.at[pl.ds(base + c * CH, CH)])
            if c + 1 < nch:
                g_prev = g_next

    return k(table, gidx)


def _edge_call(xt, feat, w1, s1, b1, w2, s2, b2, Cr, Nb=128, P=8):
    B, N, C = xt.shape
    K = _K
    G = Nb // P

    def body(xt_ref, ft_ref, w1_ref, s1_ref, b1_ref, w2_ref, s2_ref, b2_ref,
             out_ref):
        rels = []
        for b in range(B):
            xr = xt_ref[b]
            rels.append(ft_ref[b] - xr[:, None, :])
        mean = rels[0]
        for b in range(1, B):
            mean = mean + rels[b]
        mean = mean * jnp.float32(1.0 / B)
        for b in range(B):
            rel = rels[b]
            relm = rel - mean
            mk = jnp.mean(rel, axis=1, keepdims=True)
            var = jnp.sum((rel - mk) ** 2, axis=1) / jnp.float32(K - 1)
            relvar = rel * (1.0 - var)[:, None, :]
            rel2 = rel.reshape(Nb * K, C)
            relm2 = relm.reshape(Nb * K, C)
            covs = []
            for g in range(G):
                lo = g * P * K
                m = lax.dot_general(
                    rel2[lo:lo + P * K], relm2[lo:lo + P * K],
                    (((1,), (1,)), ((), ())),
                    preferred_element_type=jnp.float32)
                for p in range(P):
                    covs.append(m[p * K:(p + 1) * K, p * K:(p + 1) * K])
            cov = jnp.concatenate(covs, axis=0)
            xrb = jnp.broadcast_to(xt_ref[b][:, None, :Cr],
                                   (Nb, K, Cr)).reshape(Nb * K, Cr)
            rv = relvar[:, :, :Cr].reshape(Nb * K, Cr)
            e = jnp.concatenate([xrb, rv, cov], axis=1)
            z = lax.dot(e, w1_ref[...],
                        preferred_element_type=jnp.float32)
            z = _lrelu(z * s1_ref[...] + b1_ref[...])
            z2 = lax.dot(z, w2_ref[...],
                         preferred_element_type=jnp.float32)
            z2 = _lrelu(z2 * s2_ref[...] + b2_ref[...])
            out_ref[b] = jnp.max(z2.reshape(Nb, K, 64), axis=1)

    return pl.pallas_call(
        body,
        grid=(N // Nb,),
        in_specs=[
            pl.BlockSpec((B, Nb, C), lambda nb: (0, nb, 0)),
            pl.BlockSpec((B, Nb, K, C), lambda nb: (0, nb, 0, 0)),
            pl.BlockSpec(w1.shape, lambda nb: (0, 0)),
            pl.BlockSpec(s1.shape, lambda nb: (0, 0)),
            pl.BlockSpec(b1.shape, lambda nb: (0, 0)),
            pl.BlockSpec(w2.shape, lambda nb: (0, 0)),
            pl.BlockSpec(s2.shape, lambda nb: (0, 0)),
            pl.BlockSpec(b2.shape, lambda nb: (0, 0)),
        ],
        out_specs=pl.BlockSpec((B, Nb, 64), lambda nb: (0, nb, 0)),
        out_shape=jax.ShapeDtypeStruct((B, N, 64), jnp.float32),
    )(xt, feat, w1, s1, b1, w2, s2, b2)


def _head1_call(rc, drc, w7r, w7d, b7, Nb=512):
    B, N, _ = rc.shape

    def body(rc_ref, drc_ref, w7r_ref, w7d_ref, b7_ref, out_ref):
        nb = pl.program_id(1)
        z = (lax.dot(rc_ref[0], w7r_ref[...],
                     preferred_element_type=jnp.float32)
             + lax.dot(drc_ref[0], w7d_ref[...],
                       preferred_element_type=jnp.float32)
             + b7_ref[...])
        m = jnp.max(_lrelu(z), axis=0, keepdims=True)

        @pl.when(nb == 0)
        def _():
            out_ref[0] = m

        @pl.when(nb != 0)
        def _():
            out_ref[0] = jnp.maximum(out_ref[0], m)

    return pl.pallas_call(
        body,
        grid=(B, N // Nb),
        in_specs=[
            pl.BlockSpec((1, Nb, 64), lambda b, nb: (b, nb, 0)),
            pl.BlockSpec((1, Nb, 64), lambda b, nb: (b, nb, 0)),
            pl.BlockSpec(w7r.shape, lambda b, nb: (0, 0)),
            pl.BlockSpec(w7d.shape, lambda b, nb: (0, 0)),
            pl.BlockSpec(b7.shape, lambda b, nb: (0, 0)),
        ],
        out_specs=pl.BlockSpec((1, 1, 1024), lambda b, nb: (b, 0, 0)),
        out_shape=jax.ShapeDtypeStruct((B, 1, 1024), jnp.float32),
    )(rc, drc, w7r, w7d, b7)


def _head2_call(rc, drc, hv, lv_in, w8, b8, w9h, w9l, w9r, w9d, b9,
                w10, b10, w11, b11, w12, Nb=512):
    B, N, _ = rc.shape

    def body(rc_ref, drc_ref, hv_ref, l_ref, w8_ref, b8_ref, w9h_ref,
             w9l_ref, w9r_ref, w9d_ref, b9_ref, w10_ref, b10_ref, w11_ref,
             b11_ref, w12_ref, out_ref, bias_s):
        nb = pl.program_id(1)

        @pl.when(nb == 0)
        def _():
            lv = _lrelu(lax.dot(l_ref[0], w8_ref[...],
                                preferred_element_type=jnp.float32)
                        + b8_ref[...])
            bias_s[...] = (lax.dot(hv_ref[0], w9h_ref[...],
                                   preferred_element_type=jnp.float32)
                           + lax.dot(lv, w9l_ref[...],
                                     preferred_element_type=jnp.float32)
                           + b9_ref[...])

        z = _lrelu(lax.dot(rc_ref[0], w9r_ref[...],
                           preferred_element_type=jnp.float32)
                   + lax.dot(drc_ref[0], w9d_ref[...],
                             preferred_element_type=jnp.float32)
                   + bias_s[...])
        z = _lrelu(lax.dot(z, w10_ref[...],
                           preferred_element_type=jnp.float32) + b10_ref[...])
        z = _lrelu(lax.dot(z, w11_ref[...],
                           preferred_element_type=jnp.float32) + b11_ref[...])
        out_ref[0] = lax.dot(z, w12_ref[...],
                             preferred_element_type=jnp.float32)

    full = lambda a: pl.BlockSpec(a.shape, lambda b, nb: (0,) * a.ndim)
    return pl.pallas_call(
        body,
        grid=(B, N // Nb),
        in_specs=[
            pl.BlockSpec((1, Nb, 64), lambda b, nb: (b, nb, 0)),
            pl.BlockSpec((1, Nb, 64), lambda b, nb: (b, nb, 0)),
            pl.BlockSpec((1, 1, 1024), lambda b, nb: (b, 0, 0)),
            pl.BlockSpec((1, 1, 16), lambda b, nb: (b, 0, 0)),
            full(w8), full(b8), full(w9h), full(w9l), full(w9r), full(w9d),
            full(b9), full(w10), full(b10), full(w11), full(b11), full(w12),
        ],
        out_specs=pl.BlockSpec((1, Nb, 50), lambda b, nb: (b, nb, 0)),
        out_shape=jax.ShapeDtypeStruct((B, N, 50), jnp.float32),
        scratch_shapes=[pltpu.VMEM((1, 512), jnp.float32)],
    )(rc, drc, hv, lv_in, w8, b8, w9h, w9l, w9r, w9d, b9, w10, b10, w11,
      b11, w12)


def kernel(x, l, W3, W4, W5, W6, W7, W8, W9, W10, W11, W12,
           g3, b3, g4, b4, g5, b5, g6, b6, g7, b7, g8, b8,
           g9, b9, g10, b10, g11, b11):
    B, C0, N = x.shape
    K = _K
    s = lambda g: g / jnp.sqrt(jnp.float32(1.0 + 1e-5))
    row = lambda v: v.reshape(1, -1)

    W7f = W7 * s(g7)[:, None]
    w7r = W7f[:, 0:64].T
    w7d = W7f[:, 64:128].T
    w8 = (W8 * s(g8)[:, None]).T
    W9f = W9 * s(g9)[:, None]
    w9h = W9f[:, 0:1024].T
    w9l = W9f[:, 1024:1088].T
    w9r = W9f[:, 1088:1152].T
    w9d = W9f[:, 1152:1216].T
    w10 = (W10 * s(g10)[:, None]).T
    w11 = (W11 * s(g11)[:, None]).T
    w12 = W12.T

    xt = jnp.transpose(x, (0, 2, 1))
    xtp = jnp.pad(xt, ((0, 0), (0, 0), (0, 13)))
    xp = jnp.pad(x, ((0, 0), (0, 13), (0, 0)))
    xx1 = jnp.sum(x ** 2, axis=1, keepdims=True)
    idx1 = _topk_call(xtp, xp, xx1, jnp.transpose(xx1, (0, 2, 1)))
    feat1 = _gather_rows(xtp.reshape(B * N, 16), idx1.reshape(-1), 16)
    rc = _edge_call(xtp, feat1.reshape(B, N, K, 16),
                    W3.T, row(s(g3)), row(b3), W4.T, row(s(g4)), row(b4),
                    3)

    rcc = jnp.transpose(rc, (0, 2, 1))
    xx2 = jnp.sum(rcc ** 2, axis=1, keepdims=True)
    idx2 = _topk_call(rc, rcc, xx2, jnp.transpose(xx2, (0, 2, 1)))
    feat2 = _gather_rows(rc.reshape(B * N, 64), idx2.reshape(-1), 64)
    drc = _edge_call(rc, feat2.reshape(B, N, K, 64),
                     W5.T, row(s(g5)), row(b5), W6.T, row(s(g6)), row(b6),
                     64)

    hv = _head1_call(rc, drc, w7r, w7d, row(b7))
    out = _head2_call(rc, drc, hv, l.reshape(B, 1, 16),
                      w8, row(b8), w9h, w9l, w9r, w9d, row(b9),
                      w10, row(b10), w11, row(b11), w12)
    return jnp.transpose(out, (0, 2, 1))
